# baseline (speedup 1.0000x reference)
import functools

import jax
import jax.numpy as jnp
from jax import lax
from jax.experimental import pallas as pl
from jax.experimental.pallas import tpu as pltpu
from jax.experimental.pallas import tpu_sc as plsc

N = 10000
D = 128
G = 64
NCORE = 2
NSUB = 16
NW = NCORE * NSUB
K = 128
NPAD = 10112
ROWS = NPAD // NSUB
BN = 2000
NB = N // BN

_mesh = functools.partial(
    plsc.VectorSubcoreMesh,
    core_axis_name="c", subcore_axis_name="s",
    num_cores=NCORE, num_subcores=NSUB)

_P = jax.lax.Precision.HIGHEST


def _sc_deg(dstp, ones16, zdeg):
    nch = dstp.shape[1]

    @functools.partial(
        pl.kernel,
        out_type=jax.ShapeDtypeStruct((NCORE, NPAD, 16), jnp.float32),
        mesh=_mesh(),
        compiler_params=pltpu.CompilerParams(use_tc_tiling_on_sc=False),
        scratch_types=[
            pltpu.VMEM((nch, K), jnp.int32),
            pltpu.VMEM((K, 16), jnp.float32),
            pltpu.VMEM_SHARED((NPAD, 16), jnp.float32),
        ],
    )
    def k(dst_hbm, ones_hbm, z_hbm, out_hbm, dst_v, ones_v, acc_sh):
        c = lax.axis_index("c")
        s = lax.axis_index("s")
        wid = c * NSUB + s
        pltpu.sync_copy(z_hbm.at[pl.ds(s * ROWS, ROWS)],
                        acc_sh.at[pl.ds(s * ROWS, ROWS)])
        pltpu.sync_copy(dst_hbm.at[wid], dst_v)
        pltpu.sync_copy(ones_hbm, ones_v)
        plsc.subcore_barrier()

        @pl.loop(0, nch)
        def _(j):
            pltpu.sync_copy(ones_v, acc_sh.at[dst_v.at[j]], add=True)

        plsc.subcore_barrier()
        pltpu.sync_copy(acc_sh.at[pl.ds(s * ROWS, ROWS)],
                        out_hbm.at[c, pl.ds(s * ROWS, ROWS)])

    return k(dstp, ones16, zdeg)


PIPE = 4
DH = D // 2


GC = 16
YR = N // NSUB


def _sc_edge(y2, src16, dst16, zacc):
    nch = src16.shape[1]
    assert nch % GC == 0 and GC % PIPE == 0
    ngr = nch // GC
    assert ngr % 2 == 0

    @functools.partial(
        pl.kernel,
        out_type=jax.ShapeDtypeStruct((NCORE, NPAD, DH), jnp.float32),
        mesh=_mesh(),
        compiler_params=pltpu.CompilerParams(use_tc_tiling_on_sc=False),
        scratch_types=[
            pltpu.VMEM((2, 2, GC, K), jnp.int32),
            pltpu.VMEM((PIPE, K, DH), jnp.float32),
            pltpu.VMEM_SHARED((N, DH), jnp.float32),
            pltpu.VMEM_SHARED((NPAD, DH), jnp.float32),
        ] + [pltpu.SemaphoreType.DMA] * (2 * PIPE + 2),
    )
    def k(y_hbm, src_hbm, dst_hbm, z_hbm, out_hbm, idx_v, buf, ytab_sh,
          acc_sh, *sems):
        gsem = sems[:PIPE]
        ssem = sems[PIPE:2 * PIPE]
        isem = sems[2 * PIPE:]
        c = lax.axis_index("c")
        s = lax.axis_index("s")
        pltpu.sync_copy(y_hbm.at[pl.ds(c * N + s * YR, YR)],
                        ytab_sh.at[pl.ds(s * YR, YR)])
        pltpu.sync_copy(z_hbm.at[pl.ds(s * ROWS, ROWS)],
                        acc_sh.at[pl.ds(s * ROWS, ROWS)])

        def idx_load(g, sl):
            pltpu.async_copy(src_hbm.at[s, pl.ds(g * GC, GC)],
                             idx_v.at[sl, 0], isem[sl])
            pltpu.async_copy(dst_hbm.at[s, pl.ds(g * GC, GC)],
                             idx_v.at[sl, 1], isem[sl])

        def idx_wait(g, sl):
            pltpu.make_async_copy(src_hbm.at[s, pl.ds(g * GC, GC)],
                                  idx_v.at[sl, 0], isem[sl]).wait()
            pltpu.make_async_copy(dst_hbm.at[s, pl.ds(g * GC, GC)],
                                  idx_v.at[sl, 1], isem[sl]).wait()

        idx_load(0, 0)
        plsc.subcore_barrier()

        def group(g, sl):
            def gather(u, t):
                pltpu.async_copy(ytab_sh.at[idx_v.at[sl, 0, u]], buf.at[t],
                                 gsem[t])

            def gather_wait(u, t):
                pltpu.make_async_copy(ytab_sh.at[idx_v.at[sl, 0, u]],
                                      buf.at[t], gsem[t]).wait()

            def scat(u, t):
                pltpu.async_copy(buf.at[t], acc_sh.at[idx_v.at[sl, 1, u]],
                                 ssem[t], add=True)

            def scat_wait(u, t):
                pltpu.make_async_copy(buf.at[t],
                                      acc_sh.at[idx_v.at[sl, 1, u]],
                                      ssem[t]).wait()

            idx_wait(g, sl)

            @pl.when(g + 1 < ngr)
            def _():
                idx_load(g + 1, 1 - sl)

            for t in range(PIPE):
                gather(t, t)

            DRAIN = PIPE // 2

            @pl.loop(0, GC, step=PIPE)
            def _(u):
                for t in range(PIPE):
                    uu = u + t
                    gather_wait(uu, t)
                    scat(uu, t)
                    tp = (t - DRAIN) % PIPE
                    up = uu - DRAIN

                    @pl.when((up >= 0) & (up + PIPE < GC))
                    def _():
                        scat_wait(up, tp)
                        gather(up + PIPE, tp)

            for t in range(PIPE):
                scat_wait(GC - PIPE + t, t)

        @pl.loop(0, ngr, step=2)
        def _(go):
            for sl in range(2):
                group(go + sl, sl)

        plsc.subcore_barrier()
        pltpu.sync_copy(acc_sh.at[pl.ds(s * ROWS, ROWS)],
                        out_hbm.at[c, pl.ds(s * ROWS, ROWS)])

    return k(y2, src16, dst16, zacc)


def _dinv_of(degp_ref):
    return lax.rsqrt(degp_ref[0][:, 0:1] + degp_ref[1][:, 0:1] + 1.0)


def _wsplit(W):
    return W.reshape(D, 2, DH).transpose(1, 0, 2)


def _ymm_body(h_ref, w_ref, degp_ref, out_ref):
    xw = jnp.dot(h_ref[...], w_ref[0],
                 preferred_element_type=jnp.float32, precision=_P)
    out_ref[...] = _dinv_of(degp_ref) * xw


def _tc_ymm(h, W, degp):
    return pl.pallas_call(
        _ymm_body,
        grid=(NB, 2),
        in_specs=[
            pl.BlockSpec((BN, D), lambda i, c: (i, 0)),
            pl.BlockSpec((1, D, DH), lambda i, c: (c, 0, 0)),
            pl.BlockSpec((NCORE, BN, 16), lambda i, c: (0, i, 0)),
        ],
        out_specs=pl.BlockSpec((BN, DH), lambda i, c: (c * NB + i, 0)),
        out_shape=jax.ShapeDtypeStruct((2 * N, DH), jnp.float32),
    )(h, _wsplit(W), degp)


def _bn_relu(t, st_ref, g_ref, be_ref):
    mean = st_ref[0:1, :] * (1.0 / N)
    msq = st_ref[1:2, :] * (1.0 / N)
    var = msq - mean * mean
    rstd = lax.rsqrt(var + 1e-5)
    return jnp.maximum((t - mean) * (rstd * g_ref[...]) + be_ref[...], 0.0)


def _t_of(acc_ref, yl_ref, yr_ref, degp_ref, b_ref):
    a = jnp.concatenate([acc_ref[0] + yl_ref[...], acc_ref[1] + yr_ref[...]],
                        axis=1)
    return _dinv_of(degp_ref) * a + b_ref[...], _dinv_of(degp_ref)


def _statsnorm_body(acc_ref, yl_ref, yr_ref, degp_ref, b_ref, g_ref, be_ref,
                    w_ref, out_ref, st_ref):
    p = pl.program_id(0)
    i = pl.program_id(1)
    c = pl.program_id(2)
    t, dinv = _t_of(acc_ref, yl_ref, yr_ref, degp_ref, b_ref)

    @pl.when((p == 0) & (c == 0))
    def _():
        @pl.when(i == 0)
        def _():
            st_ref[...] = jnp.zeros_like(st_ref)

        st_ref[0:1, :] += jnp.sum(t, axis=0, keepdims=True)
        st_ref[1:2, :] += jnp.sum(t * t, axis=0, keepdims=True)

    @pl.when(p == 1)
    def _():
        h = _bn_relu(t, st_ref, g_ref, be_ref)
        xw = jnp.dot(h, w_ref[0],
                     preferred_element_type=jnp.float32, precision=_P)
        out_ref[...] = dinv * xw


def _tc_statsnorm(acc, y2, degp, b, g, be, Wn):
    return pl.pallas_call(
        _statsnorm_body,
        grid=(2, NB, 2),
        in_specs=[
            pl.BlockSpec((NCORE, BN, DH), lambda p, i, c: (0, i, 0)),
            pl.BlockSpec((BN, DH), lambda p, i, c: (i, 0)),
            pl.BlockSpec((BN, DH), lambda p, i, c: (NB + i, 0)),
            pl.BlockSpec((NCORE, BN, 16), lambda p, i, c: (0, i, 0)),
            pl.BlockSpec((1, D), lambda p, i, c: (0, 0)),
            pl.BlockSpec((1, D), lambda p, i, c: (0, 0)),
            pl.BlockSpec((1, D), lambda p, i, c: (0, 0)),
            pl.BlockSpec((1, D, DH), lambda p, i, c: (c, 0, 0)),
        ],
        out_specs=pl.BlockSpec(
            (BN, DH), lambda p, i, c: (p * (c * NB + i) + (1 - p) * (2 * NB - 1), 0)),
        out_shape=jax.ShapeDtypeStruct((2 * N, DH), jnp.float32),
        scratch_shapes=[pltpu.VMEM((2, D), jnp.float32)],
    )(acc, y2, y2, degp, b, g, be, _wsplit(Wn))


def _statshead_body(acc_ref, yl_ref, yr_ref, degp_ref, b_ref, g_ref, be_ref,
                    batch_ref, wh1_ref, bh1_ref, wh2_ref, bh2_ref, out_ref,
                    st_ref, ps_ref, cnt_ref):
    p = pl.program_id(0)
    i = pl.program_id(1)
    t, _ = _t_of(acc_ref, yl_ref, yr_ref, degp_ref, b_ref)

    @pl.when(p == 0)
    def _():
        @pl.when(i == 0)
        def _():
            st_ref[...] = jnp.zeros_like(st_ref)

        st_ref[0:1, :] += jnp.sum(t, axis=0, keepdims=True)
        st_ref[1:2, :] += jnp.sum(t * t, axis=0, keepdims=True)

    @pl.when(p == 1)
    def _():
        @pl.when(i == 0)
        def _():
            ps_ref[...] = jnp.zeros_like(ps_ref)
            cnt_ref[...] = jnp.zeros_like(cnt_ref)

        h = _bn_relu(t, st_ref, g_ref, be_ref)
        bt = batch_ref[0]
        oh = (lax.broadcasted_iota(jnp.int32, (G, BN), 0) == bt)
        oh = oh.astype(jnp.float32)
        ps_ref[...] += jnp.dot(oh, h, preferred_element_type=jnp.float32,
                               precision=_P)
        cnt_ref[...] += jnp.broadcast_to(
            jnp.sum(oh, axis=1, keepdims=True), (G, D))

        @pl.when(i == NB - 1)
        def _():
            cnt = jnp.maximum(cnt_ref[:, 0:1], 1.0)
            pooled = ps_ref[...] / cnt
            hh = jnp.maximum(
                jnp.dot(pooled, wh1_ref[...],
                        preferred_element_type=jnp.float32, precision=_P)
                + bh1_ref[...], 0.0)
            out_ref[...] = (jnp.sum(hh * wh2_ref[...], axis=1, keepdims=True)
                            + bh2_ref[...])


def _tc_statshead(acc, y2, degp, b, g, be, batch3, Wh1, bh1, Wh2r, bh2r):
    return pl.pallas_call(
        _statshead_body,
        grid=(2, NB),
        in_specs=[
            pl.BlockSpec((NCORE, BN, DH), lambda p, i: (0, i, 0)),
            pl.BlockSpec((BN, DH), lambda p, i: (i, 0)),
            pl.BlockSpec((BN, DH), lambda p, i: (NB + i, 0)),
            pl.BlockSpec((NCORE, BN, 16), lambda p, i: (0, i, 0)),
            pl.BlockSpec((1, D), lambda p, i: (0, 0)),
            pl.BlockSpec((1, D), lambda p, i: (0, 0)),
            pl.BlockSpec((1, D), lambda p, i: (0, 0)),
            pl.BlockSpec((1, 1, BN), lambda p, i: (i, 0, 0)),
            pl.BlockSpec((D, G), lambda p, i: (0, 0)),
            pl.BlockSpec((1, G), lambda p, i: (0, 0)),
            pl.BlockSpec((1, G), lambda p, i: (0, 0)),
            pl.BlockSpec((1, 1), lambda p, i: (0, 0)),
        ],
        out_specs=pl.BlockSpec((G, 1), lambda p, i: (0, 0)),
        out_shape=jax.ShapeDtypeStruct((G, 1), jnp.float32),
        scratch_shapes=[
            pltpu.VMEM((2, D), jnp.float32),
            pltpu.VMEM((G, D), jnp.float32),
            pltpu.VMEM((G, D), jnp.float32),
        ],
    )(acc, y2, y2, degp, b, g, be, batch3, Wh1, bh1, Wh2r, bh2r)


def kernel(x, edge_index, batch, W0, b0, g0, be0, W1, b1, g1, be1,
           W2, b2, g2, be2, Wh1, bh1, Wh2, bh2):
    E = edge_index.shape[1]
    src = edge_index[0]
    dst = edge_index[1]

    nch32 = -(-E // (NW * K))
    pad32 = NW * nch32 * K - E
    dstp = jnp.concatenate([dst, jnp.full((pad32,), N, jnp.int32)])
    dstp = dstp.reshape(NW, nch32, K)

    nch = -(-E // (NSUB * K))
    nch += (-nch) % (2 * GC)
    pad = NSUB * nch * K - E
    src16 = jnp.concatenate([src, jnp.zeros((pad,), jnp.int32)])
    src16 = src16.reshape(NSUB, nch, K)
    dst16 = jnp.concatenate([dst, jnp.full((pad,), N, jnp.int32)])
    dst16 = dst16.reshape(NSUB, nch, K)

    zacc = jnp.zeros((NPAD, DH), jnp.float32)
    zdeg = jnp.zeros((NPAD, 16), jnp.float32)
    ones16 = jnp.ones((K, 16), jnp.float32)
    batch3 = batch.reshape(NB, 1, BN)

    b0r = b0.reshape(1, D); b1r = b1.reshape(1, D); b2r = b2.reshape(1, D)
    g0r = g0.reshape(1, D); g1r = g1.reshape(1, D); g2r = g2.reshape(1, D)
    be0r = be0.reshape(1, D); be1r = be1.reshape(1, D); be2r = be2.reshape(1, D)
    bh1r = bh1.reshape(1, G); bh2r = bh2.reshape(1, 1)
    wh2r = Wh2.reshape(1, G)

    degp = _sc_deg(dstp, ones16, zdeg)

    y = _tc_ymm(x, W0, degp)
    for (bi, gi, bei, Wn) in ((b0r, g0r, be0r, W1), (b1r, g1r, be1r, W2)):
        acc = _sc_edge(y, src16, dst16, zacc)
        y = _tc_statsnorm(acc, y, degp, bi, gi, bei, Wn)

    acc = _sc_edge(y, src16, dst16, zacc)
    out = _tc_statshead(acc, y, degp, b2r, g2r, be2r, batch3, Wh1, bh1r,
                        wh2r, bh2r)
    return jnp.squeeze(out, axis=-1)


# speedup vs baseline: 1.0644x; 1.0644x over previous
import functools

import jax
import jax.numpy as jnp
from jax import lax
from jax.experimental import pallas as pl
from jax.experimental.pallas import tpu as pltpu
from jax.experimental.pallas import tpu_sc as plsc

N = 10000
D = 128
G = 64
NCORE = 2
NSUB = 16
NW = NCORE * NSUB
K = 128
NPAD = 10112
ROWS = NPAD // NSUB
BN = 2000
NB = N // BN

_mesh = functools.partial(
    plsc.VectorSubcoreMesh,
    core_axis_name="c", subcore_axis_name="s",
    num_cores=NCORE, num_subcores=NSUB)

_P = None
_PX = jax.lax.Precision.HIGHEST


def _sc_deg(dstp, ones16, zdeg):
    nch = dstp.shape[1]

    @functools.partial(
        pl.kernel,
        out_type=jax.ShapeDtypeStruct((NCORE, NPAD, 16), jnp.float32),
        mesh=_mesh(),
        compiler_params=pltpu.CompilerParams(use_tc_tiling_on_sc=False),
        scratch_types=[
            pltpu.VMEM((nch, K), jnp.int32),
            pltpu.VMEM((K, 16), jnp.float32),
            pltpu.VMEM_SHARED((NPAD, 16), jnp.float32),
        ],
    )
    def k(dst_hbm, ones_hbm, z_hbm, out_hbm, dst_v, ones_v, acc_sh):
        c = lax.axis_index("c")
        s = lax.axis_index("s")
        wid = c * NSUB + s
        pltpu.sync_copy(z_hbm.at[pl.ds(s * ROWS, ROWS)],
                        acc_sh.at[pl.ds(s * ROWS, ROWS)])
        pltpu.sync_copy(dst_hbm.at[wid], dst_v)
        pltpu.sync_copy(ones_hbm, ones_v)
        plsc.subcore_barrier()

        @pl.loop(0, nch)
        def _(j):
            pltpu.sync_copy(ones_v, acc_sh.at[dst_v.at[j]], add=True)

        plsc.subcore_barrier()
        pltpu.sync_copy(acc_sh.at[pl.ds(s * ROWS, ROWS)],
                        out_hbm.at[c, pl.ds(s * ROWS, ROWS)])

    return k(dstp, ones16, zdeg)


PIPE = 4
DH = D // 2


GC = 16
YR = N // NSUB


def _sc_edge(y2, src16, dst16, zacc):
    nch = src16.shape[1]
    assert nch % GC == 0 and GC % PIPE == 0
    ngr = nch // GC
    assert ngr % 2 == 0

    @functools.partial(
        pl.kernel,
        out_type=jax.ShapeDtypeStruct((NCORE, NPAD, DH), jnp.float32),
        mesh=_mesh(),
        compiler_params=pltpu.CompilerParams(use_tc_tiling_on_sc=False),
        scratch_types=[
            pltpu.VMEM((2, 2, GC, K), jnp.int32),
            pltpu.VMEM((PIPE, K, DH), jnp.float32),
            pltpu.VMEM_SHARED((N, DH), jnp.float32),
            pltpu.VMEM_SHARED((NPAD, DH), jnp.float32),
        ] + [pltpu.SemaphoreType.DMA] * (2 * PIPE + 2),
    )
    def k(y_hbm, src_hbm, dst_hbm, z_hbm, out_hbm, idx_v, buf, ytab_sh,
          acc_sh, *sems):
        gsem = sems[:PIPE]
        ssem = sems[PIPE:2 * PIPE]
        isem = sems[2 * PIPE:]
        c = lax.axis_index("c")
        s = lax.axis_index("s")
        pltpu.sync_copy(y_hbm.at[pl.ds(c * N + s * YR, YR)],
                        ytab_sh.at[pl.ds(s * YR, YR)])
        pltpu.sync_copy(z_hbm.at[pl.ds(s * ROWS, ROWS)],
                        acc_sh.at[pl.ds(s * ROWS, ROWS)])

        def idx_load(g, sl):
            pltpu.async_copy(src_hbm.at[s, pl.ds(g * GC, GC)],
                             idx_v.at[sl, 0], isem[sl])
            pltpu.async_copy(dst_hbm.at[s, pl.ds(g * GC, GC)],
                             idx_v.at[sl, 1], isem[sl])

        def idx_wait(g, sl):
            pltpu.make_async_copy(src_hbm.at[s, pl.ds(g * GC, GC)],
                                  idx_v.at[sl, 0], isem[sl]).wait()
            pltpu.make_async_copy(dst_hbm.at[s, pl.ds(g * GC, GC)],
                                  idx_v.at[sl, 1], isem[sl]).wait()

        idx_load(0, 0)
        plsc.subcore_barrier()

        def group(g, sl):
            def gather(u, t):
                pltpu.async_copy(ytab_sh.at[idx_v.at[sl, 0, u]], buf.at[t],
                                 gsem[t])

            def gather_wait(u, t):
                pltpu.make_async_copy(ytab_sh.at[idx_v.at[sl, 0, u]],
                                      buf.at[t], gsem[t]).wait()

            def scat(u, t):
                pltpu.async_copy(buf.at[t], acc_sh.at[idx_v.at[sl, 1, u]],
                                 ssem[t], add=True)

            def scat_wait(u, t):
                pltpu.make_async_copy(buf.at[t],
                                      acc_sh.at[idx_v.at[sl, 1, u]],
                                      ssem[t]).wait()

            idx_wait(g, sl)

            @pl.when(g + 1 < ngr)
            def _():
                idx_load(g + 1, 1 - sl)

            for t in range(PIPE):
                gather(t, t)

            DRAIN = PIPE // 2

            @pl.loop(0, GC, step=PIPE)
            def _(u):
                for t in range(PIPE):
                    uu = u + t
                    gather_wait(uu, t)
                    scat(uu, t)
                    tp = (t - DRAIN) % PIPE
                    up = uu - DRAIN

                    @pl.when((up >= 0) & (up + PIPE < GC))
                    def _():
                        scat_wait(up, tp)
                        gather(up + PIPE, tp)

            for t in range(PIPE):
                scat_wait(GC - PIPE + t, t)

        @pl.loop(0, ngr, step=2)
        def _(go):
            for sl in range(2):
                group(go + sl, sl)

        plsc.subcore_barrier()
        pltpu.sync_copy(acc_sh.at[pl.ds(s * ROWS, ROWS)],
                        out_hbm.at[c, pl.ds(s * ROWS, ROWS)])

    return k(y2, src16, dst16, zacc)


def _wsplit(W):
    return W.reshape(D, 2, DH).transpose(1, 0, 2)


def _ymm_body(h_ref, w_ref, degp_ref, out_ref):
    xw = jnp.dot(h_ref[...], w_ref[0],
                 preferred_element_type=jnp.float32, precision=_P)
    out_ref[...] = lax.rsqrt(degp_ref[0][:, 0:1] + degp_ref[1][:, 0:1]
                             + 1.0) * xw


def _tc_ymm(h, W, degp):
    return pl.pallas_call(
        _ymm_body,
        grid=(NB, 2),
        in_specs=[
            pl.BlockSpec((BN, D), lambda i, c: (i, 0)),
            pl.BlockSpec((1, D, DH), lambda i, c: (c, 0, 0)),
            pl.BlockSpec((NCORE, BN, 16), lambda i, c: (0, i, 0)),
        ],
        out_specs=pl.BlockSpec((BN, DH), lambda i, c: (c * NB + i, 0)),
        out_shape=jax.ShapeDtypeStruct((2 * N, DH), jnp.float32),
    )(h, _wsplit(W), degp)


def _stats_body(acc_ref, yl_ref, yr_ref, degp_ref, b_ref, t_ref, st_ref,
                stm_ref, stv_ref):
    i = pl.program_id(0)
    dinv = lax.rsqrt(degp_ref[0][:, 0:1] + degp_ref[1][:, 0:1] + 1.0)
    a = jnp.concatenate([acc_ref[0] + yl_ref[...], acc_ref[1] + yr_ref[...]],
                        axis=1)
    t = dinv * a + b_ref[...]
    t_ref[...] = t

    m = jnp.sum(t, axis=0, keepdims=True) * (1.0 / BN)
    d0 = t - m
    stm_ref[pl.ds(i, 1), :] = m
    stv_ref[pl.ds(i, 1), :] = jnp.sum(d0 * d0, axis=0, keepdims=True)

    @pl.when(i == NB - 1)
    def _():
        mm = jnp.sum(stm_ref[0:NB], axis=0, keepdims=True) * (1.0 / NB)
        dm = stm_ref[0:NB] - mm
        var = (jnp.sum(stv_ref[0:NB], axis=0, keepdims=True) * (1.0 / N)
               + jnp.sum(dm * dm, axis=0, keepdims=True) * (1.0 / NB))
        st_ref[0:1, :] = mm
        st_ref[1:2, :] = lax.rsqrt(var + 1e-5)


def _tc_stats(acc, y2, degp, b):
    return pl.pallas_call(
        _stats_body,
        grid=(NB,),
        in_specs=[
            pl.BlockSpec((NCORE, BN, DH), lambda i: (0, i, 0)),
            pl.BlockSpec((BN, DH), lambda i: (i, 0)),
            pl.BlockSpec((BN, DH), lambda i: (NB + i, 0)),
            pl.BlockSpec((NCORE, BN, 16), lambda i: (0, i, 0)),
            pl.BlockSpec((1, D), lambda i: (0, 0)),
        ],
        out_specs=[
            pl.BlockSpec((BN, D), lambda i: (i, 0)),
            pl.BlockSpec((2, D), lambda i: (0, 0)),
        ],
        out_shape=[
            jax.ShapeDtypeStruct((N, D), jnp.float32),
            jax.ShapeDtypeStruct((2, D), jnp.float32),
        ],
        scratch_shapes=[
            pltpu.VMEM((8, D), jnp.float32),
            pltpu.VMEM((8, D), jnp.float32),
        ],
    )(acc, y2, y2, degp, b)


def _bn_relu(t, st_ref, g_ref, be_ref):
    mean = st_ref[0:1, :]
    rstd = st_ref[1:2, :]
    return jnp.maximum((t - mean) * (rstd * g_ref[...]) + be_ref[...], 0.0)


def _normmm_body(t_ref, st_ref, g_ref, be_ref, w_ref, degp_ref, out_ref):
    h = _bn_relu(t_ref[...], st_ref, g_ref, be_ref)
    xw = jnp.dot(h, w_ref[0],
                 preferred_element_type=jnp.float32, precision=_P)
    out_ref[...] = lax.rsqrt(degp_ref[0][:, 0:1] + degp_ref[1][:, 0:1]
                             + 1.0) * xw


def _tc_normmm(t, st, g, be, Wn, degp):
    return pl.pallas_call(
        _normmm_body,
        grid=(NB, 2),
        in_specs=[
            pl.BlockSpec((BN, D), lambda i, c: (i, 0)),
            pl.BlockSpec((2, D), lambda i, c: (0, 0)),
            pl.BlockSpec((1, D), lambda i, c: (0, 0)),
            pl.BlockSpec((1, D), lambda i, c: (0, 0)),
            pl.BlockSpec((1, D, DH), lambda i, c: (c, 0, 0)),
            pl.BlockSpec((NCORE, BN, 16), lambda i, c: (0, i, 0)),
        ],
        out_specs=pl.BlockSpec((BN, DH), lambda i, c: (c * NB + i, 0)),
        out_shape=jax.ShapeDtypeStruct((2 * N, DH), jnp.float32),
    )(t, st, g, be, _wsplit(Wn), degp)


def _head_body(t_ref, st_ref, g_ref, be_ref, batch_ref, wh1_ref, bh1_ref,
               wh2_ref, bh2_ref, out_ref, ps_ref, cnt_ref):
    i = pl.program_id(0)

    @pl.when(i == 0)
    def _():
        ps_ref[...] = jnp.zeros_like(ps_ref)
        cnt_ref[...] = jnp.zeros_like(cnt_ref)

    h = _bn_relu(t_ref[...], st_ref, g_ref, be_ref)
    bt = batch_ref[0]
    oh = (lax.broadcasted_iota(jnp.int32, (G, BN), 0) == bt)
    oh = oh.astype(jnp.float32)
    ps_ref[...] += jnp.dot(oh, h, preferred_element_type=jnp.float32,
                           precision=_PX)
    cnt_ref[...] += jnp.broadcast_to(
        jnp.sum(oh, axis=1, keepdims=True), (G, D))

    @pl.when(i == NB - 1)
    def _():
        cnt = jnp.maximum(cnt_ref[:, 0:1], 1.0)
        pooled = ps_ref[...] / cnt
        hh = jnp.maximum(
            jnp.dot(pooled, wh1_ref[...],
                    preferred_element_type=jnp.float32, precision=_P)
            + bh1_ref[...], 0.0)
        out_ref[...] = jnp.dot(hh, wh2_ref[...],
                               preferred_element_type=jnp.float32,
                               precision=_P) + bh2_ref[...]


def _tc_head(t, st, g, be, batch3, Wh1, bh1, Wh2, bh2r):
    return pl.pallas_call(
        _head_body,
        grid=(NB,),
        in_specs=[
            pl.BlockSpec((BN, D), lambda i: (i, 0)),
            pl.BlockSpec((2, D), lambda i: (0, 0)),
            pl.BlockSpec((1, D), lambda i: (0, 0)),
            pl.BlockSpec((1, D), lambda i: (0, 0)),
            pl.BlockSpec((1, 1, BN), lambda i: (i, 0, 0)),
            pl.BlockSpec((D, G), lambda i: (0, 0)),
            pl.BlockSpec((1, G), lambda i: (0, 0)),
            pl.BlockSpec((G, 1), lambda i: (0, 0)),
            pl.BlockSpec((1, 1), lambda i: (0, 0)),
        ],
        out_specs=pl.BlockSpec((G, 1), lambda i: (0, 0)),
        out_shape=jax.ShapeDtypeStruct((G, 1), jnp.float32),
        scratch_shapes=[
            pltpu.VMEM((G, D), jnp.float32),
            pltpu.VMEM((G, D), jnp.float32),
        ],
    )(t, st, g, be, batch3, Wh1, bh1, Wh2, bh2r)


def kernel(x, edge_index, batch, W0, b0, g0, be0, W1, b1, g1, be1,
           W2, b2, g2, be2, Wh1, bh1, Wh2, bh2):
    E = edge_index.shape[1]
    src = edge_index[0]
    dst = edge_index[1]

    nch32 = -(-E // (NW * K))
    pad32 = NW * nch32 * K - E
    dstp = jnp.concatenate([dst, jnp.full((pad32,), N, jnp.int32)])
    dstp = dstp.reshape(NW, nch32, K)

    nch = -(-E // (NSUB * K))
    nch += (-nch) % (2 * GC)
    pad = NSUB * nch * K - E
    src16 = jnp.concatenate([src, jnp.zeros((pad,), jnp.int32)])
    src16 = src16.reshape(NSUB, nch, K)
    dst16 = jnp.concatenate([dst, jnp.full((pad,), N, jnp.int32)])
    dst16 = dst16.reshape(NSUB, nch, K)

    zacc = jnp.zeros((NPAD, DH), jnp.float32)
    zdeg = jnp.zeros((NPAD, 16), jnp.float32)
    ones16 = jnp.ones((K, 16), jnp.float32)
    batch3 = batch.reshape(NB, 1, BN)

    b0r = b0.reshape(1, D); b1r = b1.reshape(1, D); b2r = b2.reshape(1, D)
    g0r = g0.reshape(1, D); g1r = g1.reshape(1, D); g2r = g2.reshape(1, D)
    be0r = be0.reshape(1, D); be1r = be1.reshape(1, D); be2r = be2.reshape(1, D)
    bh1r = bh1.reshape(1, G); bh2r = bh2.reshape(1, 1)

    degp = _sc_deg(dstp, ones16, zdeg)

    y = _tc_ymm(x, W0, degp)
    for (bi, gi, bei, Wn) in ((b0r, g0r, be0r, W1), (b1r, g1r, be1r, W2)):
        acc = _sc_edge(y, src16, dst16, zacc)
        t, st = _tc_stats(acc, y, degp, bi)
        y = _tc_normmm(t, st, gi, bei, Wn, degp)

    acc = _sc_edge(y, src16, dst16, zacc)
    t, st = _tc_stats(acc, y, degp, b2r)
    out = _tc_head(t, st, g2r, be2r, batch3, Wh1, bh1r, Wh2, bh2r)
    return jnp.squeeze(out, axis=-1)
